# baseline (speedup 1.0000x reference)
import functools

import jax
import jax.numpy as jnp
from jax import lax
from jax.experimental import pallas as pl
from jax.experimental.pallas import tpu as pltpu
from jax.experimental.pallas import tpu_sc as plsc

F32 = jnp.float32
N = 10000
E = 320000
HID = 128
NB = 32
RB = 1000
NRB = N // RB

NC, NS = 2, 16
NW = NC * NS
EPW = E // NW
CH = 40
NCHUNK = EPW // CH
RPS = 632
NP = NS * RPS
ZR = 8


def _ln(x, g, b):
    m = jnp.mean(x, axis=-1, keepdims=True)
    xc = x - m
    v = jnp.mean(xc * xc, axis=-1, keepdims=True)
    return xc * jax.lax.rsqrt(v + 1e-5) * g + b


def _dot(a, b):
    return jnp.dot(a, b, preferred_element_type=F32)


EB = 4000
NEB = E // EB


def _mean_body(ea_ref, We0, We1, We2, ma_ref, ep0_ref, ep1_ref, ep2_ref):
    i = pl.program_id(0)
    ea = ea_ref[...]
    ep0_ref[...] = _dot(ea, We0[...])
    ep1_ref[...] = _dot(ea, We1[...])
    ep2_ref[...] = _dot(ea, We2[...])
    part = jnp.sum(ea, axis=0, keepdims=True) * (1.0 / E)

    @pl.when(i == 0)
    def _():
        ma_ref[...] = part

    @pl.when(i > 0)
    def _():
        ma_ref[...] = ma_ref[...] + part


def _mean_eproj(edge_attr, p):
    full = lambda s: pl.BlockSpec(s, lambda g: (0, 0))
    row = lambda s: pl.BlockSpec(s, lambda g: (g, 0))
    return pl.pallas_call(
        _mean_body,
        grid=(NEB,),
        in_specs=[row((EB, 8)), full((8, HID)), full((8, HID)), full((8, HID))],
        out_specs=[full((1, 8)), row((EB, HID)), row((EB, HID)), row((EB, HID))],
        out_shape=[jax.ShapeDtypeStruct((1, 8), F32)]
        + [jax.ShapeDtypeStruct((E, HID), F32)] * 3,
    )(edge_attr, p['gat0_We'], p['gat1_We'], p['gat2_We'])


def _t0_body(x_ref, ma_ref, pW, pb, lg, lb, We0, We1, We2, Wl, bl, Wr, br,
             h_ref, xl_ref, xr_ref, em0_ref, em1_ref, em2_ref):
    i = pl.program_id(0)
    h = _dot(x_ref[...], pW[...]) + pb[...]
    h = jnp.maximum(_ln(h, lg[...], lb[...]), 0.0)
    h_ref[...] = h
    xl_ref[...] = _dot(h, Wl[...]) + bl[...]
    xr_ref[...] = _dot(h, Wr[...]) + br[...]

    @pl.when(i == 0)
    def _():
        ma = ma_ref[...]
        em0_ref[...] = _dot(ma, We0[...])
        em1_ref[...] = _dot(ma, We1[...])
        em2_ref[...] = _dot(ma, We2[...])


def _prologue(x, ma, p):
    full = lambda s: pl.BlockSpec(s, lambda i: (0, 0))
    row = lambda s: pl.BlockSpec(s, lambda i: (i, 0))
    r1 = lambda v: v.reshape(1, -1)
    return pl.pallas_call(
        _t0_body,
        grid=(NRB,),
        in_specs=[row((RB, HID)), full((1, 8)), full((HID, HID)), full((1, HID)),
                  full((1, HID)), full((1, HID)),
                  full((8, HID)), full((8, HID)), full((8, HID)),
                  full((HID, HID)), full((1, HID)), full((HID, HID)), full((1, HID))],
        out_specs=[row((RB, HID)), row((RB, HID)), row((RB, HID)),
                   full((1, HID)), full((1, HID)), full((1, HID))],
        out_shape=[jax.ShapeDtypeStruct((N, HID), F32)] * 3
        + [jax.ShapeDtypeStruct((1, HID), F32)] * 3,
    )(x, ma, p['proj_W'], r1(p['proj_b']), r1(p['proj_ln_g']), r1(p['proj_ln_b']),
      p['gat0_We'], p['gat1_We'], p['gat2_We'],
      p['gat0_Wl'], r1(p['gat0_bl']), p['gat0_Wr'], r1(p['gat0_br']))


@functools.cache
def _edge_kernel():
  mesh = plsc.VectorSubcoreMesh(core_axis_name="c", subcore_axis_name="s",
                                num_cores=NC, num_subcores=NS)

  @functools.partial(
    pl.kernel,
    out_type=[jax.ShapeDtypeStruct((NC, NP, HID), F32),
              jax.ShapeDtypeStruct((NC, NP, 16), F32)],
    mesh=mesh,
    scratch_types=[
        [pltpu.VMEM((CH,), jnp.int32)] * 2,
        [pltpu.VMEM((CH,), jnp.int32)] * 2,
        [pltpu.VMEM((CH, HID), F32)] * 2,
        [pltpu.VMEM((CH, HID), F32)] * 2,
        [pltpu.VMEM((CH, HID), F32)] * 2,
        pltpu.VMEM((CH, 16), F32),
        pltpu.VMEM((8, 16), F32),
        pltpu.VMEM_SHARED((NP, HID), F32),
        pltpu.VMEM_SHARED((NP, 16), F32),
        [pltpu.SemaphoreType.DMA] * 2,
        [pltpu.SemaphoreType.DMA] * 2,
    ],
    compiler_params=pltpu.CompilerParams(use_tc_tiling_on_sc=False),
  )
  def _edge_body(xl_hbm, xr_hbm, src_hbm, dst_hbm, ep_hbm, att_hbm,
                 zU_hbm, zD_hbm,
                 U_out, D_out,
                 srcv, dstv, epv, xlv, xrv, exv, attv,
                 Usp, Dsp, semL, semG):
    cid = lax.axis_index("c")
    sid = lax.axis_index("s")
    wid = cid * NS + sid
    lane = lax.iota(jnp.int32, 16)
    zv = jnp.zeros((16,), F32)

    pltpu.sync_copy(att_hbm, attv)

    r0 = sid * RPS
    pltpu.sync_copy(zU_hbm, Usp.at[pl.ds(r0, RPS)])
    pltpu.sync_copy(zD_hbm, Dsp.at[pl.ds(r0, RPS)])
    plsc.subcore_barrier()

    def _lin_copies(c, b):
        off = wid * EPW + c * CH
        return ((src_hbm.at[pl.ds(off, CH)], srcv[b]),
                (dst_hbm.at[pl.ds(off, CH)], dstv[b]),
                (ep_hbm.at[pl.ds(off, CH)], epv[b]))

    def _issue_lin(c, b):
        for s, d in _lin_copies(c, b):
            pltpu.async_copy(s, d, semL[b])

    def _wait_lin(c, b):
        for s, d in _lin_copies(c, b):
            pltpu.make_async_copy(s, d, semL[b]).wait()

    def _issue_gat(b):
        pltpu.async_copy(xl_hbm.at[srcv[b]], xlv[b], semG[b])
        pltpu.async_copy(xr_hbm.at[dstv[b]], xrv[b], semG[b])

    def _wait_gat(b):
        pltpu.make_async_copy(xl_hbm.at[srcv[b]], xlv[b], semG[b]).wait()
        pltpu.make_async_copy(xr_hbm.at[dstv[b]], xrv[b], semG[b]).wait()

    def _compute_scatter(b):
        def _g(v, k):
            return v.at[lane ^ k].get(mode="promise_in_bounds")

        def _gf(v, j):
            return v.at[jnp.full((16,), j, jnp.int32)].get(
                mode="promise_in_bounds")

        def _edge(e, cc):
            tl, xls = [], []
            for d in range(8):
                sl = pl.ds(d * 16, 16)
                xld = xlv[b][e, sl]
                acc = xld + xrv[b][e, sl] + epv[b][e, sl]
                lz = jnp.maximum(acc, 0.2 * acc)
                tl.append(lz * attv[d])
                xls.append(xld)
            ev = zv
            for hp in range(4):
                a = tl[2 * hp]
                c2 = tl[2 * hp + 1]
                a = a + _g(a, 8)
                c2 = c2 + _g(c2, 8)
                m = jnp.where(lane < 8, a, _g(c2, 8))
                for k in (4, 2, 1):
                    m = m + _g(m, k)
                em_ = jnp.exp(m)
                s0 = _gf(em_, 0)
                s1 = _gf(em_, 8)
                xrv[b][e, pl.ds((2 * hp) * 16, 16)] = xls[2 * hp] * s0
                xrv[b][e, pl.ds((2 * hp + 1) * 16, 16)] = xls[2 * hp + 1] * s1
                ev = jnp.where(lane == 2 * hp, s0, ev)
                ev = jnp.where(lane == 2 * hp + 1, s1, ev)
            exv[e] = ev
            return cc
        lax.fori_loop(0, CH, _edge, 0)
        pltpu.sync_copy(xrv[b], Usp.at[dstv[b]], add=True)
        pltpu.sync_copy(exv, Dsp.at[dstv[b]], add=True)

    _issue_lin(0, 0)
    _wait_lin(0, 0)
    _issue_gat(0)
    _issue_lin(1, 1)

    def _pair(j, c):
        for b in (0, 1):
            ci = 2 * j + b
            _wait_lin(ci + 1, 1 - b)
            _issue_gat(1 - b)
            _wait_gat(b)
            _compute_scatter(b)
            _issue_lin(ci + 2, b)
        return c
    lax.fori_loop(0, (NCHUNK - 2) // 2, _pair, 0)
    _wait_lin(NCHUNK - 1, 1)
    _issue_gat(1)
    _wait_gat(0)
    _compute_scatter(0)
    _wait_gat(1)
    _compute_scatter(1)

    plsc.subcore_barrier()
    pltpu.sync_copy(Usp.at[pl.ds(r0, RPS)], U_out.at[cid, pl.ds(r0, RPS)])
    pltpu.sync_copy(Dsp.at[pl.ds(r0, RPS)], D_out.at[cid, pl.ds(r0, RPS)])

  return _edge_body


def _edge_sc(*args):
    return _edge_kernel()(*args)


def _attn_combine(Up_ref, Dp_ref, xl_ref, xr_ref, em_ref, att_ref, gb_ref,
                  Ps_ref, Pe_ref):
    xl = xl_ref[...]
    z = xl + xr_ref[...] + em_ref[...]
    lz = jnp.maximum(z, 0.2 * z)
    t = lz * att_ref[...]
    ex = jnp.exp(_dot(t, Ps_ref[...]))
    exe = _dot(ex, Pe_ref[...])
    U = Up_ref[0] + Up_ref[1] + xl * exe
    D8 = Dp_ref[0][:, 0:8] + Dp_ref[1][:, 0:8] + ex
    De = _dot(D8, Pe_ref[...])
    return U / (De + 1e-16) + gb_ref[...]


def _te01_body(Up_ref, Dp_ref, xl_ref, xr_ref, em_ref, att_ref, hp_ref,
               lg, lb, gb_ref, Ps_ref, Pe_ref, Wl, bl, Wr, br,
               h_ref, xln_ref, xrn_ref, *, residual):
    out = _attn_combine(Up_ref, Dp_ref, xl_ref, xr_ref, em_ref, att_ref, gb_ref,
                        Ps_ref, Pe_ref)
    hn = _ln(out, lg[...], lb[...])
    if residual:
        hn = hn + hp_ref[...]
    hn = jnp.maximum(hn, 0.0)
    h_ref[...] = hn
    xln_ref[...] = _dot(hn, Wl[...]) + bl[...]
    xrn_ref[...] = _dot(hn, Wr[...]) + br[...]


def _epilogue01(i, Up, Dp, xl, xr, em, att, hprev, Ps, Pe, p):
    full = lambda s: pl.BlockSpec(s, lambda g: (0, 0))
    row = lambda s: pl.BlockSpec(s, lambda g: (g, 0))
    r1 = lambda v: v.reshape(1, -1)
    j = i + 1
    return pl.pallas_call(
        functools.partial(_te01_body, residual=(i > 0)),
        grid=(NRB,),
        in_specs=[pl.BlockSpec((2, RB, HID), lambda g: (0, g, 0)),
                  pl.BlockSpec((2, RB, 16), lambda g: (0, g, 0)),
                  row((RB, HID)), row((RB, HID)), full((1, HID)), full((1, HID)),
                  row((RB, HID)), full((1, HID)), full((1, HID)), full((1, HID)),
                  full((HID, 8)), full((8, HID)),
                  full((HID, HID)), full((1, HID)), full((HID, HID)), full((1, HID))],
        out_specs=[row((RB, HID))] * 3,
        out_shape=[jax.ShapeDtypeStruct((N, HID), F32)] * 3,
    )(Up, Dp, xl, xr, em, att, hprev,
      r1(p[f'ln{i}_g']), r1(p[f'ln{i}_b']), r1(p[f'gat{i}_bias']), Ps, Pe,
      p[f'gat{j}_Wl'], r1(p[f'gat{j}_bl']), p[f'gat{j}_Wr'], r1(p[f'gat{j}_br']))


def _te2_body(Up_ref, Dp_ref, xl_ref, xr_ref, em_ref, att_ref, hp_ref,
              lg, lb, gb_ref, Ps_ref, Pe_ref,
              chW1, chb1, chW2, chb2, caW1, cab1, caW2, cab2, bat_ref,
              ch_ref, ca_ref, S_ref, cnt_ref):
    g = pl.program_id(0)
    out = _attn_combine(Up_ref, Dp_ref, xl_ref, xr_ref, em_ref, att_ref, gb_ref,
                        Ps_ref, Pe_ref)
    hn = _ln(out, lg[...], lb[...]) + hp_ref[...]
    hn = jnp.maximum(hn, 0.0)
    ch = jax.nn.sigmoid(
        _dot(jnp.maximum(_dot(hn, chW1[...]) + chb1[...], 0.0), chW2[...])
        + chb2[...])
    ca = (_dot(jnp.maximum(_dot(hn, caW1[...]) + cab1[...], 0.0), caW2[...])
          + cab2[...])
    ch_ref[...] = ch
    ca_ref[...] = ca
    bat = bat_ref[0]
    oh = jnp.where(
        lax.broadcasted_iota(jnp.int32, (NB, RB), 0) == bat, 1.0, 0.0)
    S_part = lax.dot_general(oh, hn, (((1,), (0,)), ((), ())),
                             preferred_element_type=F32)
    cnt_part = jnp.sum(oh, axis=1, keepdims=True)

    @pl.when(g == 0)
    def _():
        S_ref[...] = S_part
        cnt_ref[...] = cnt_part

    @pl.when(g > 0)
    def _():
        S_ref[...] = S_ref[...] + S_part
        cnt_ref[...] = cnt_ref[...] + cnt_part


def _epilogue2(Up, Dp, xl, xr, em, att, hprev, bat3, Ps, Pe, p):
    full = lambda s: pl.BlockSpec(s, lambda g: (0, 0))
    row = lambda s: pl.BlockSpec(s, lambda g: (g, 0))
    r1 = lambda v: v.reshape(1, -1)
    return pl.pallas_call(
        _te2_body,
        grid=(NRB,),
        in_specs=[pl.BlockSpec((2, RB, HID), lambda g: (0, g, 0)),
                  pl.BlockSpec((2, RB, 16), lambda g: (0, g, 0)),
                  row((RB, HID)), row((RB, HID)), full((1, HID)), full((1, HID)),
                  row((RB, HID)), full((1, HID)), full((1, HID)), full((1, HID)),
                  full((HID, 8)), full((8, HID)),
                  full((HID, 64)), full((1, 64)), full((64, 1)), full((1, 1)),
                  full((HID, HID)), full((1, HID)), full((HID, 9)), full((1, 9)),
                  pl.BlockSpec((1, 1, RB), lambda g: (g, 0, 0))],
        out_specs=[row((RB, 1)), row((RB, 9)),
                   full((NB, HID)), full((NB, 1))],
        out_shape=[jax.ShapeDtypeStruct((N, 1), F32),
                   jax.ShapeDtypeStruct((N, 9), F32),
                   jax.ShapeDtypeStruct((NB, HID), F32),
                   jax.ShapeDtypeStruct((NB, 1), F32)],
    )(Up, Dp, xl, xr, em, att, hprev,
      r1(p['ln2_g']), r1(p['ln2_b']), r1(p['gat2_bias']), Ps, Pe,
      p['ch_W1'], r1(p['ch_b1']), p['ch_W2'], r1(p['ch_b2']),
      p['ca_W1'], r1(p['ca_b1']), p['ca_W2'], r1(p['ca_b2']), bat3)


def _tg_body(S, cnt, WihT0, bih0, bhh0, WihT1, bih1, bhh1,
             ghW1, ghb1, ghW2, ghb2, gdW1, gdb1, gdW2, gdb2,
             gaW1, gab1, gaW2, gab2, ruW1, rub1, ruW2, rub2,
             gh_ref, gd_ref, ga_ref, rul_ref):
    g = S[...] / jnp.maximum(cnt[...], 1.0)
    inp = g
    for WT, bi, bh in ((WihT0, bih0, bhh0), (WihT1, bih1, bhh1)):
        gates = _dot(inp, WT[...]) + bi[...] + bh[...]
        i_ = jax.nn.sigmoid(gates[:, 0:256])
        g_ = jnp.tanh(gates[:, 512:768])
        o_ = jax.nn.sigmoid(gates[:, 768:1024])
        inp = o_ * jnp.tanh(i_ * g_)
    mlp = lambda W1, b1, W2, b2: (
        _dot(jnp.maximum(_dot(inp, W1[...]) + b1[...], 0.0), W2[...]) + b2[...])
    gh_ref[...] = jax.nn.sigmoid(mlp(ghW1, ghb1, ghW2, ghb2))
    gd_ref[...] = jax.nn.sigmoid(mlp(gdW1, gdb1, gdW2, gdb2))
    ga_ref[...] = mlp(gaW1, gab1, gaW2, gab2)
    rul_ref[...] = jax.nn.softplus(mlp(ruW1, rub1, ruW2, rub2))


def _global_head(S, cnt, p):
    r1 = lambda v: v.reshape(1, -1)
    return pl.pallas_call(
        _tg_body,
        out_shape=[jax.ShapeDtypeStruct((NB, 1), F32),
                   jax.ShapeDtypeStruct((NB, 1), F32),
                   jax.ShapeDtypeStruct((NB, 9), F32),
                   jax.ShapeDtypeStruct((NB, 1), F32)],
    )(S, cnt, p['lstm0_Wih'].T, r1(p['lstm0_bih']), r1(p['lstm0_bhh']),
      p['lstm1_Wih'].T, r1(p['lstm1_bih']), r1(p['lstm1_bhh']),
      p['gh_W1'], r1(p['gh_b1']), p['gh_W2'], r1(p['gh_b2']),
      p['gd_W1'], r1(p['gd_b1']), p['gd_W2'], r1(p['gd_b2']),
      p['ga_W1'], r1(p['ga_b1']), p['ga_W2'], r1(p['ga_b2']),
      p['rul_W1'], r1(p['rul_b1']), p['rul_W2'], r1(p['rul_b2']))


def kernel(x, edge_index, edge_attr, batch, params):
    p = params
    src = edge_index[0].astype(jnp.int32)
    dst = edge_index[1].astype(jnp.int32)
    bat3 = batch.astype(jnp.int32).reshape(NRB, 1, RB)

    zU = jnp.zeros((RPS, HID), F32)
    zD = jnp.zeros((RPS, 16), F32)
    Ps = jnp.repeat(jnp.eye(8, dtype=F32), 16, axis=0)
    Pe = Ps.T
    ma, ep0, ep1, ep2 = _mean_eproj(edge_attr, p)
    h, xl, xr, em0, em1, em2 = _prologue(x, ma, p)
    ems = (em0, em1, em2)
    eps = (ep0, ep1, ep2)

    for i in range(3):
        Up, Dp = _edge_sc(xl, xr, src, dst, eps[i], p[f'gat{i}_att'],
                          zU, zD)
        att = p[f'gat{i}_att'].reshape(1, HID)
        if i < 2:
            h, xl, xr = _epilogue01(i, Up, Dp, xl, xr, ems[i], att, h,
                                    Ps, Pe, p)
        else:
            ch, ca, S, cnt = _epilogue2(Up, Dp, xl, xr, ems[i], att, h, bat3,
                                        Ps, Pe, p)

    gh, gd, ga, rul = _global_head(S, cnt, p)
    return ch, ca, gh, gd, ga, rul


# speedup vs baseline: 1.1340x; 1.1340x over previous
import functools

import jax
import jax.numpy as jnp
from jax import lax
from jax.experimental import pallas as pl
from jax.experimental.pallas import tpu as pltpu
from jax.experimental.pallas import tpu_sc as plsc

F32 = jnp.float32
N = 10000
E = 320000
HID = 128
NB = 32
RB = 1000
NRB = N // RB

NC, NS = 2, 16
NW = NC * NS
EPW = E // NW
CH = 40
NCHUNK = EPW // CH
RPS = 632
NP = NS * RPS
ZR = 8


def _ln(x, g, b):
    m = jnp.mean(x, axis=-1, keepdims=True)
    xc = x - m
    v = jnp.mean(xc * xc, axis=-1, keepdims=True)
    return xc * jax.lax.rsqrt(v + 1e-5) * g + b


def _dot(a, b):
    return jnp.dot(a, b, preferred_element_type=F32)


EB = 4000
NEB = E // EB


def _mean_body(ea_ref, We0, We1, We2, ma_ref, ep0_ref, ep1_ref, ep2_ref):
    i = pl.program_id(0)
    ea = ea_ref[...]
    ep0_ref[...] = _dot(ea, We0[...])
    ep1_ref[...] = _dot(ea, We1[...])
    ep2_ref[...] = _dot(ea, We2[...])
    part = jnp.sum(ea, axis=0, keepdims=True) * (1.0 / E)

    @pl.when(i == 0)
    def _():
        ma_ref[...] = part

    @pl.when(i > 0)
    def _():
        ma_ref[...] = ma_ref[...] + part


def _mean_eproj(edge_attr, p):
    full = lambda s: pl.BlockSpec(s, lambda g: (0, 0))
    row = lambda s: pl.BlockSpec(s, lambda g: (g, 0))
    return pl.pallas_call(
        _mean_body,
        grid=(NEB,),
        in_specs=[row((EB, 8)), full((8, HID)), full((8, HID)), full((8, HID))],
        out_specs=[full((1, 8)), row((EB, HID)), row((EB, HID)), row((EB, HID))],
        out_shape=[jax.ShapeDtypeStruct((1, 8), F32)]
        + [jax.ShapeDtypeStruct((E, HID), F32)] * 3,
    )(edge_attr, p['gat0_We'], p['gat1_We'], p['gat2_We'])


def _t0_body(x_ref, ma_ref, pW, pb, lg, lb, We0, We1, We2, Wl, bl, Wr, br,
             h_ref, xl_ref, xr_ref, em0_ref, em1_ref, em2_ref):
    i = pl.program_id(0)
    h = _dot(x_ref[...], pW[...]) + pb[...]
    h = jnp.maximum(_ln(h, lg[...], lb[...]), 0.0)
    h_ref[...] = h
    xl_ref[...] = _dot(h, Wl[...]) + bl[...]
    xr_ref[...] = _dot(h, Wr[...]) + br[...]

    @pl.when(i == 0)
    def _():
        ma = ma_ref[...]
        em0_ref[...] = _dot(ma, We0[...])
        em1_ref[...] = _dot(ma, We1[...])
        em2_ref[...] = _dot(ma, We2[...])


def _prologue(x, ma, p):
    full = lambda s: pl.BlockSpec(s, lambda i: (0, 0))
    row = lambda s: pl.BlockSpec(s, lambda i: (i, 0))
    r1 = lambda v: v.reshape(1, -1)
    return pl.pallas_call(
        _t0_body,
        grid=(NRB,),
        in_specs=[row((RB, HID)), full((1, 8)), full((HID, HID)), full((1, HID)),
                  full((1, HID)), full((1, HID)),
                  full((8, HID)), full((8, HID)), full((8, HID)),
                  full((HID, HID)), full((1, HID)), full((HID, HID)), full((1, HID))],
        out_specs=[row((RB, HID)), row((RB, HID)), row((RB, HID)),
                   full((1, HID)), full((1, HID)), full((1, HID))],
        out_shape=[jax.ShapeDtypeStruct((N, HID), F32)] * 3
        + [jax.ShapeDtypeStruct((1, HID), F32)] * 3,
    )(x, ma, p['proj_W'], r1(p['proj_b']), r1(p['proj_ln_g']), r1(p['proj_ln_b']),
      p['gat0_We'], p['gat1_We'], p['gat2_We'],
      p['gat0_Wl'], r1(p['gat0_bl']), p['gat0_Wr'], r1(p['gat0_br']))


@functools.cache
def _edge_kernel():
  mesh = plsc.VectorSubcoreMesh(core_axis_name="c", subcore_axis_name="s",
                                num_cores=NC, num_subcores=NS)

  @functools.partial(
    pl.kernel,
    out_type=[jax.ShapeDtypeStruct((NC, NP, HID), F32),
              jax.ShapeDtypeStruct((NC, NP, 16), F32)],
    mesh=mesh,
    scratch_types=[
        [pltpu.VMEM((CH,), jnp.int32)] * 2,
        [pltpu.VMEM((CH,), jnp.int32)] * 2,
        [pltpu.VMEM((CH, HID), F32)] * 2,
        [pltpu.VMEM((CH, HID), F32)] * 2,
        [pltpu.VMEM((CH, HID), F32)] * 2,
        [pltpu.VMEM((CH,), jnp.int32)] * 2,
        [pltpu.VMEM((CH, 16), F32)] * 2,
        pltpu.VMEM((8, 16), F32),
        pltpu.VMEM_SHARED((NP, HID), F32),
        pltpu.VMEM_SHARED((NP, 16), F32),
        [pltpu.SemaphoreType.DMA] * 2,
        [pltpu.SemaphoreType.DMA] * 2,
        [pltpu.SemaphoreType.DMA] * 2,
    ],
    compiler_params=pltpu.CompilerParams(use_tc_tiling_on_sc=False),
  )
  def _edge_body(xl_hbm, xr_hbm, src_hbm, dst_hbm, ep_hbm, att_hbm,
                 zU_hbm, zD_hbm,
                 U_out, D_out,
                 srcv, dstv, epv, xlv, xrv, dstc, exv, attv,
                 Usp, Dsp, semL, semG, semS):
    cid = lax.axis_index("c")
    sid = lax.axis_index("s")
    wid = cid * NS + sid
    lane = lax.iota(jnp.int32, 16)
    zv = jnp.zeros((16,), F32)

    pltpu.sync_copy(att_hbm, attv)

    r0 = sid * RPS
    pltpu.sync_copy(zU_hbm, Usp.at[pl.ds(r0, RPS)])
    pltpu.sync_copy(zD_hbm, Dsp.at[pl.ds(r0, RPS)])
    plsc.subcore_barrier()

    def _lin_copies(c, b):
        off = wid * EPW + c * CH
        return ((src_hbm.at[pl.ds(off, CH)], srcv[b]),
                (dst_hbm.at[pl.ds(off, CH)], dstv[b]),
                (ep_hbm.at[pl.ds(off, CH)], epv[b]))

    def _issue_lin(c, b):
        for s, d in _lin_copies(c, b):
            pltpu.async_copy(s, d, semL[b])

    def _wait_lin(c, b):
        for s, d in _lin_copies(c, b):
            pltpu.make_async_copy(s, d, semL[b]).wait()

    def _issue_gat(b):
        pltpu.async_copy(xl_hbm.at[srcv[b]], xlv[b], semG[b])
        pltpu.async_copy(xr_hbm.at[dstv[b]], xrv[b], semG[b])

    def _wait_gat(b):
        pltpu.make_async_copy(xl_hbm.at[srcv[b]], xlv[b], semG[b]).wait()
        pltpu.make_async_copy(xr_hbm.at[dstv[b]], xrv[b], semG[b]).wait()

    def _compute(b):
        def _edge(e, cc):
            exs = []
            for d in range(8):
                sl = pl.ds(d * 16, 16)
                acc = xlv[b][e, sl] + xrv[b][e, sl] + epv[b][e, sl]
                lz = jnp.maximum(acc, 0.2 * acc)
                s = lz * attv[d]
                for k in (8, 4, 2, 1):
                    s = s + s.at[lane ^ k].get(mode="promise_in_bounds")
                exs.append(jnp.exp(s))
            ev = zv
            for d in range(8):
                sl = pl.ds(d * 16, 16)
                ev = jnp.where(lane == d, exs[d], ev)
                xrv[b][e, sl] = xlv[b][e, sl] * exs[d]
            exv[b][e] = ev
            return cc
        lax.fori_loop(0, CH, _edge, 0)
        for q0 in (0, 16, CH - 16):
            dstc[b][pl.ds(q0, 16)] = dstv[b][pl.ds(q0, 16)]

    def _sct_copies(b):
        return ((xrv[b], Usp.at[dstc[b]]),
                (exv[b], Dsp.at[dstc[b]]))

    def _issue_sct(b):
        for s, d in _sct_copies(b):
            pltpu.async_copy(s, d, semS[b], add=True)

    def _wait_sct(b):
        for s, d in _sct_copies(b):
            pltpu.make_async_copy(s, d, semS[b]).wait()

    _issue_lin(0, 0)
    _wait_lin(0, 0)
    _issue_gat(0)
    _issue_lin(1, 1)

    def _pair(j, c):
        for b in (0, 1):
            ci = 2 * j + b
            _wait_lin(ci + 1, 1 - b)
            _issue_gat(1 - b)
            _wait_gat(b)

            @pl.when(ci >= 2)
            def _():
                _wait_sct(b)
            _compute(b)
            _issue_sct(b)
            _issue_lin(ci + 2, b)
        return c
    lax.fori_loop(0, (NCHUNK - 2) // 2, _pair, 0)
    _wait_lin(NCHUNK - 1, 1)
    _issue_gat(1)
    _wait_gat(0)
    _wait_sct(0)
    _compute(0)
    _issue_sct(0)
    _wait_gat(1)
    _wait_sct(1)
    _compute(1)
    _issue_sct(1)
    _wait_sct(0)
    _wait_sct(1)

    plsc.subcore_barrier()
    pltpu.sync_copy(Usp.at[pl.ds(r0, RPS)], U_out.at[cid, pl.ds(r0, RPS)])
    pltpu.sync_copy(Dsp.at[pl.ds(r0, RPS)], D_out.at[cid, pl.ds(r0, RPS)])

  return _edge_body


def _edge_sc(*args):
    return _edge_kernel()(*args)


def _attn_combine(Up_ref, Dp_ref, xl_ref, xr_ref, em_ref, att_ref, gb_ref,
                  Ps_ref, Pe_ref):
    xl = xl_ref[...]
    z = xl + xr_ref[...] + em_ref[...]
    lz = jnp.maximum(z, 0.2 * z)
    t = lz * att_ref[...]
    ex = jnp.exp(_dot(t, Ps_ref[...]))
    exe = _dot(ex, Pe_ref[...])
    U = Up_ref[0] + Up_ref[1] + xl * exe
    D8 = Dp_ref[0][:, 0:8] + Dp_ref[1][:, 0:8] + ex
    De = _dot(D8, Pe_ref[...])
    return U / (De + 1e-16) + gb_ref[...]


def _te01_body(Up_ref, Dp_ref, xl_ref, xr_ref, em_ref, att_ref, hp_ref,
               lg, lb, gb_ref, Ps_ref, Pe_ref, Wl, bl, Wr, br,
               h_ref, xln_ref, xrn_ref, *, residual):
    out = _attn_combine(Up_ref, Dp_ref, xl_ref, xr_ref, em_ref, att_ref, gb_ref,
                        Ps_ref, Pe_ref)
    hn = _ln(out, lg[...], lb[...])
    if residual:
        hn = hn + hp_ref[...]
    hn = jnp.maximum(hn, 0.0)
    h_ref[...] = hn
    xln_ref[...] = _dot(hn, Wl[...]) + bl[...]
    xrn_ref[...] = _dot(hn, Wr[...]) + br[...]


def _epilogue01(i, Up, Dp, xl, xr, em, att, hprev, Ps, Pe, p):
    full = lambda s: pl.BlockSpec(s, lambda g: (0, 0))
    row = lambda s: pl.BlockSpec(s, lambda g: (g, 0))
    r1 = lambda v: v.reshape(1, -1)
    j = i + 1
    return pl.pallas_call(
        functools.partial(_te01_body, residual=(i > 0)),
        grid=(NRB,),
        in_specs=[pl.BlockSpec((2, RB, HID), lambda g: (0, g, 0)),
                  pl.BlockSpec((2, RB, 16), lambda g: (0, g, 0)),
                  row((RB, HID)), row((RB, HID)), full((1, HID)), full((1, HID)),
                  row((RB, HID)), full((1, HID)), full((1, HID)), full((1, HID)),
                  full((HID, 8)), full((8, HID)),
                  full((HID, HID)), full((1, HID)), full((HID, HID)), full((1, HID))],
        out_specs=[row((RB, HID))] * 3,
        out_shape=[jax.ShapeDtypeStruct((N, HID), F32)] * 3,
    )(Up, Dp, xl, xr, em, att, hprev,
      r1(p[f'ln{i}_g']), r1(p[f'ln{i}_b']), r1(p[f'gat{i}_bias']), Ps, Pe,
      p[f'gat{j}_Wl'], r1(p[f'gat{j}_bl']), p[f'gat{j}_Wr'], r1(p[f'gat{j}_br']))


def _te2_body(Up_ref, Dp_ref, xl_ref, xr_ref, em_ref, att_ref, hp_ref,
              lg, lb, gb_ref, Ps_ref, Pe_ref,
              chW1, chb1, chW2, chb2, caW1, cab1, caW2, cab2, bat_ref,
              ch_ref, ca_ref, S_ref, cnt_ref):
    g = pl.program_id(0)
    out = _attn_combine(Up_ref, Dp_ref, xl_ref, xr_ref, em_ref, att_ref, gb_ref,
                        Ps_ref, Pe_ref)
    hn = _ln(out, lg[...], lb[...]) + hp_ref[...]
    hn = jnp.maximum(hn, 0.0)
    ch = jax.nn.sigmoid(
        _dot(jnp.maximum(_dot(hn, chW1[...]) + chb1[...], 0.0), chW2[...])
        + chb2[...])
    ca = (_dot(jnp.maximum(_dot(hn, caW1[...]) + cab1[...], 0.0), caW2[...])
          + cab2[...])
    ch_ref[...] = ch
    ca_ref[...] = ca
    bat = bat_ref[0]
    oh = jnp.where(
        lax.broadcasted_iota(jnp.int32, (NB, RB), 0) == bat, 1.0, 0.0)
    S_part = lax.dot_general(oh, hn, (((1,), (0,)), ((), ())),
                             preferred_element_type=F32)
    cnt_part = jnp.sum(oh, axis=1, keepdims=True)

    @pl.when(g == 0)
    def _():
        S_ref[...] = S_part
        cnt_ref[...] = cnt_part

    @pl.when(g > 0)
    def _():
        S_ref[...] = S_ref[...] + S_part
        cnt_ref[...] = cnt_ref[...] + cnt_part


def _epilogue2(Up, Dp, xl, xr, em, att, hprev, bat3, Ps, Pe, p):
    full = lambda s: pl.BlockSpec(s, lambda g: (0, 0))
    row = lambda s: pl.BlockSpec(s, lambda g: (g, 0))
    r1 = lambda v: v.reshape(1, -1)
    return pl.pallas_call(
        _te2_body,
        grid=(NRB,),
        in_specs=[pl.BlockSpec((2, RB, HID), lambda g: (0, g, 0)),
                  pl.BlockSpec((2, RB, 16), lambda g: (0, g, 0)),
                  row((RB, HID)), row((RB, HID)), full((1, HID)), full((1, HID)),
                  row((RB, HID)), full((1, HID)), full((1, HID)), full((1, HID)),
                  full((HID, 8)), full((8, HID)),
                  full((HID, 64)), full((1, 64)), full((64, 1)), full((1, 1)),
                  full((HID, HID)), full((1, HID)), full((HID, 9)), full((1, 9)),
                  pl.BlockSpec((1, 1, RB), lambda g: (g, 0, 0))],
        out_specs=[row((RB, 1)), row((RB, 9)),
                   full((NB, HID)), full((NB, 1))],
        out_shape=[jax.ShapeDtypeStruct((N, 1), F32),
                   jax.ShapeDtypeStruct((N, 9), F32),
                   jax.ShapeDtypeStruct((NB, HID), F32),
                   jax.ShapeDtypeStruct((NB, 1), F32)],
    )(Up, Dp, xl, xr, em, att, hprev,
      r1(p['ln2_g']), r1(p['ln2_b']), r1(p['gat2_bias']), Ps, Pe,
      p['ch_W1'], r1(p['ch_b1']), p['ch_W2'], r1(p['ch_b2']),
      p['ca_W1'], r1(p['ca_b1']), p['ca_W2'], r1(p['ca_b2']), bat3)


def _tg_body(S, cnt, WihT0, bih0, bhh0, WihT1, bih1, bhh1,
             ghW1, ghb1, ghW2, ghb2, gdW1, gdb1, gdW2, gdb2,
             gaW1, gab1, gaW2, gab2, ruW1, rub1, ruW2, rub2,
             gh_ref, gd_ref, ga_ref, rul_ref):
    g = S[...] / jnp.maximum(cnt[...], 1.0)
    inp = g
    for WT, bi, bh in ((WihT0, bih0, bhh0), (WihT1, bih1, bhh1)):
        gates = _dot(inp, WT[...]) + bi[...] + bh[...]
        i_ = jax.nn.sigmoid(gates[:, 0:256])
        g_ = jnp.tanh(gates[:, 512:768])
        o_ = jax.nn.sigmoid(gates[:, 768:1024])
        inp = o_ * jnp.tanh(i_ * g_)
    mlp = lambda W1, b1, W2, b2: (
        _dot(jnp.maximum(_dot(inp, W1[...]) + b1[...], 0.0), W2[...]) + b2[...])
    gh_ref[...] = jax.nn.sigmoid(mlp(ghW1, ghb1, ghW2, ghb2))
    gd_ref[...] = jax.nn.sigmoid(mlp(gdW1, gdb1, gdW2, gdb2))
    ga_ref[...] = mlp(gaW1, gab1, gaW2, gab2)
    rul_ref[...] = jax.nn.softplus(mlp(ruW1, rub1, ruW2, rub2))


def _global_head(S, cnt, p):
    r1 = lambda v: v.reshape(1, -1)
    return pl.pallas_call(
        _tg_body,
        out_shape=[jax.ShapeDtypeStruct((NB, 1), F32),
                   jax.ShapeDtypeStruct((NB, 1), F32),
                   jax.ShapeDtypeStruct((NB, 9), F32),
                   jax.ShapeDtypeStruct((NB, 1), F32)],
    )(S, cnt, p['lstm0_Wih'].T, r1(p['lstm0_bih']), r1(p['lstm0_bhh']),
      p['lstm1_Wih'].T, r1(p['lstm1_bih']), r1(p['lstm1_bhh']),
      p['gh_W1'], r1(p['gh_b1']), p['gh_W2'], r1(p['gh_b2']),
      p['gd_W1'], r1(p['gd_b1']), p['gd_W2'], r1(p['gd_b2']),
      p['ga_W1'], r1(p['ga_b1']), p['ga_W2'], r1(p['ga_b2']),
      p['rul_W1'], r1(p['rul_b1']), p['rul_W2'], r1(p['rul_b2']))


def kernel(x, edge_index, edge_attr, batch, params):
    p = params
    src = edge_index[0].astype(jnp.int32)
    dst = edge_index[1].astype(jnp.int32)
    bat3 = batch.astype(jnp.int32).reshape(NRB, 1, RB)

    zU = jnp.zeros((RPS, HID), F32)
    zD = jnp.zeros((RPS, 16), F32)
    Ps = jnp.repeat(jnp.eye(8, dtype=F32), 16, axis=0)
    Pe = Ps.T
    ma, ep0, ep1, ep2 = _mean_eproj(edge_attr, p)
    h, xl, xr, em0, em1, em2 = _prologue(x, ma, p)
    ems = (em0, em1, em2)
    eps = (ep0, ep1, ep2)

    for i in range(3):
        Up, Dp = _edge_sc(xl, xr, src, dst, eps[i], p[f'gat{i}_att'],
                          zU, zD)
        att = p[f'gat{i}_att'].reshape(1, HID)
        if i < 2:
            h, xl, xr = _epilogue01(i, Up, Dp, xl, xr, ems[i], att, h,
                                    Ps, Pe, p)
        else:
            ch, ca, S, cnt = _epilogue2(Up, Dp, xl, xr, ems[i], att, h, bat3,
                                        Ps, Pe, p)

    gh, gd, ga, rul = _global_head(S, cnt, p)
    return ch, ca, gh, gd, ga, rul
